# Initial kernel scaffold; baseline (speedup 1.0000x reference)
import functools

import jax
import jax.numpy as jnp
from jax import lax
from jax.experimental import pallas as pl
from jax.experimental.pallas import tpu as pltpu
from jax.experimental.pallas import tpu_sc as plsc

B = 16384
F = 26
VOCAB = 100000
D = 16

NC = 2
NS = 16
NW = NC * NS
BPW = B // NW

_mesh = plsc.VectorSubcoreMesh(core_axis_name="c", subcore_axis_name="s")


@functools.partial(
    pl.kernel,
    out_type=jax.ShapeDtypeStruct((B, D), jnp.float32),
    mesh=_mesh,
    scratch_types=[
        pltpu.VMEM((F, BPW), jnp.int32),
        pltpu.VMEM((BPW, D), jnp.float32),
        pltpu.SemaphoreType.DMA,
    ],
)
def _encode(xt_hbm, tables_hbm, out_hbm, idx_v, acc_v, sem):
    wid = lax.axis_index("s") * NC + lax.axis_index("c")
    base = wid * BPW
    pltpu.sync_copy(xt_hbm.at[:, pl.ds(base, BPW)], idx_v)
    pltpu.async_copy(tables_hbm.at[0].at[idx_v.at[0]], acc_v, sem).wait()
    copies = [
        pltpu.async_copy(tables_hbm.at[f].at[idx_v.at[f]], acc_v, sem, add=True)
        for f in range(1, F)
    ]
    for c in copies:
        c.wait()
    pltpu.sync_copy(acc_v, out_hbm.at[pl.ds(base, BPW), :])


def kernel(x, tables):
    xt = x.T
    return _encode(xt, tables)


# speedup vs baseline: 1.0467x; 1.0467x over previous
import functools

import jax
import jax.numpy as jnp
from jax import lax
from jax.experimental import pallas as pl
from jax.experimental.pallas import tpu as pltpu
from jax.experimental.pallas import tpu_sc as plsc

B = 16384
F = 26
VOCAB = 100000
D = 16

NC = 2
NS = 16
NW = NC * NS
BPW = B // NW

_mesh = plsc.VectorSubcoreMesh(core_axis_name="c", subcore_axis_name="s")


@functools.partial(
    pl.kernel,
    out_type=jax.ShapeDtypeStruct((B, D), jnp.float32),
    mesh=_mesh,
    scratch_types=[
        pltpu.VMEM((F, BPW), jnp.int32),
        pltpu.VMEM((BPW, D), jnp.float32),
        pltpu.SemaphoreType.DMA,
    ],
    compiler_params=pltpu.CompilerParams(use_tc_tiling_on_sc=False),
)
def _encode(xt_hbm, tables_hbm, out_hbm, idx_v, acc_v, sem):
    wid = lax.axis_index("s") * NC + lax.axis_index("c")
    base = wid * BPW
    pltpu.sync_copy(xt_hbm.at[:, pl.ds(base, BPW)], idx_v)
    pltpu.async_copy(tables_hbm.at[0].at[idx_v.at[0]], acc_v, sem).wait()
    copies = [
        pltpu.async_copy(tables_hbm.at[f].at[idx_v.at[f]], acc_v, sem, add=True)
        for f in range(1, F)
    ]
    for c in copies:
        c.wait()
    pltpu.sync_copy(acc_v, out_hbm.at[pl.ds(base, BPW), :])


def kernel(x, tables):
    xt = x.T
    return _encode(xt, tables)
